# baseline (device time: 21806 ns/iter reference)
import jax
import jax.numpy as jnp
from jax import lax
from jax.experimental import pallas as pl
from jax.experimental.pallas import tpu as pltpu

N_DEV = 4


def _all_reduce(x):
    T, D = x.shape
    C = T // N_DEV

    def body(x_ref, out_ref, rs_ref, send_sems, recv_sems):
        my = lax.axis_index("i")

        barrier = pltpu.get_barrier_semaphore()
        for p in range(N_DEV - 1):
            peer = (my + 1 + p) % N_DEV
            pl.semaphore_signal(
                barrier, inc=1, device_id=(peer,),
                device_id_type=pl.DeviceIdType.MESH,
            )
        pl.semaphore_wait(barrier, N_DEV - 1)

        phase1 = []
        for p in range(N_DEV - 1):
            peer = (my + 1 + p) % N_DEV
            rdma = pltpu.make_async_remote_copy(
                src_ref=x_ref.at[pl.ds(peer * C, C), :],
                dst_ref=rs_ref.at[2 - p],
                send_sem=send_sems.at[p],
                recv_sem=recv_sems.at[2 - p],
                device_id=(peer,),
                device_id_type=pl.DeviceIdType.MESH,
            )
            rdma.start()
            phase1.append(rdma)
        for rdma in phase1:
            rdma.wait()

        acc = x_ref[pl.ds(my * C, C), :]
        for q in range(N_DEV - 1):
            acc = acc + rs_ref[q]
        out_ref[pl.ds(my * C, C), :] = acc

        phase2 = []
        for p in range(N_DEV - 1):
            peer = (my + 1 + p) % N_DEV
            rdma = pltpu.make_async_remote_copy(
                src_ref=out_ref.at[pl.ds(my * C, C), :],
                dst_ref=out_ref.at[pl.ds(my * C, C), :],
                send_sem=send_sems.at[3 + p],
                recv_sem=recv_sems.at[3 + (2 - p)],
                device_id=(peer,),
                device_id_type=pl.DeviceIdType.MESH,
            )
            rdma.start()
            phase2.append(rdma)
        for rdma in phase2:
            rdma.wait()

    return pl.pallas_call(
        body,
        out_shape=jax.ShapeDtypeStruct((T, D), x.dtype),
        in_specs=[pl.BlockSpec(memory_space=pltpu.VMEM)],
        out_specs=pl.BlockSpec(memory_space=pltpu.VMEM),
        scratch_shapes=[
            pltpu.VMEM((N_DEV - 1, C, D), x.dtype),
            pltpu.SemaphoreType.DMA((2 * (N_DEV - 1),)),
            pltpu.SemaphoreType.DMA((2 * (N_DEV - 1),)),
        ],
        compiler_params=pltpu.CompilerParams(collective_id=0),
    )(x)


def kernel(ids, E):
    V_per, _ = E.shape
    my = lax.axis_index("i")
    local = ids - my * V_per
    in_range = (local >= 0) & (local < V_per)
    safe = jnp.where(in_range, local, 0)
    partial = jnp.where(
        in_range[:, None], jnp.take(E, safe, axis=0), jnp.float32(0)
    )
    return _all_reduce(partial)
